# baseline (speedup 1.0000x reference)
import jax
import jax.numpy as jnp
from jax.experimental import pallas as pl
from jax.experimental.pallas import tpu as pltpu

_VS = (2, 4, 8, 32)
_VPAD = 128
_D = 512
_BN = 1000
_SEGS = tuple((32 * s, 32 * s + v) for s, v in enumerate(_VS))
_NEG = -1e30
_DIMS_RR = (((1,), (1,)), ((), ()))
_DIMS_CC = (((0,), (0,)), ((), ()))


def _gcu_body(x_ref, a0_ref, s0_ref, g0_ref, a1_ref, s1_ref, g1_ref,
              a2_ref, s2_ref, g2_ref, a3_ref, s3_ref, g3_ref,
              o_ref, q_ref, qtx_ref, qs_ref, z2_ref,
              wpk_ref, spk_ref, nh_ref, wi_ref, t3_ref):
    c = pl.program_id(0)
    i = pl.program_id(1)

    @pl.when((c == 0) & (i == 0))
    def _prep():
        spk_ref[...] = jnp.ones_like(spk_ref)
        wpk_ref[...] = jnp.zeros_like(wpk_ref)
        for (lo, hi), a_ref, s_ref in ((_SEGS[0], a0_ref, s0_ref),
                                       (_SEGS[1], a1_ref, s1_ref),
                                       (_SEGS[2], a2_ref, s2_ref),
                                       (_SEGS[3], a3_ref, s3_ref)):
            wpk_ref[lo:hi, :] = a_ref[...]
            spk_ref[lo:hi, :] = jnp.abs(s_ref[...]) + 1e-4
        sig = spk_ref[...]
        w = wpk_ref[...]
        inv2 = 1.0 / (sig * sig)
        nh_ref[...] = -0.5 * inv2
        wi_ref[...] = w * inv2
        t3 = jax.lax.dot_general(
            jnp.ones((8, _D), jnp.float32), -0.5 * (w * w) * inv2, _DIMS_RR,
            preferred_element_type=jnp.float32)
        lane = jax.lax.broadcasted_iota(jnp.int32, (8, _VPAD), 1)
        within = lane % 32
        group = lane // 32
        vlim = jnp.where(group == 0, _VS[0],
                         jnp.where(group == 1, _VS[1],
                                   jnp.where(group == 2, _VS[2], _VS[3])))
        t3_ref[...] = jnp.where(within >= vlim, _NEG, t3)
        qtx_ref[...] = jnp.zeros_like(qtx_ref)
        qs_ref[...] = jnp.zeros_like(qs_ref)

    @pl.when(c == 0)
    def _assign():
        x = x_ref[...]
        t1 = jax.lax.dot_general(x * x, nh_ref[...], _DIMS_RR,
                                 preferred_element_type=jnp.float32)
        t2 = jax.lax.dot_general(x, wi_ref[...], _DIMS_RR,
                                 preferred_element_type=jnp.float32)
        neg = t1 + t2 + t3_ref[0:1, :]
        lane = jax.lax.broadcasted_iota(jnp.int32, neg.shape, 1)
        mval = jnp.full_like(neg, 1e30)
        for lo, hi in _SEGS:
            m = (lane >= lo) & (lane < hi)
            t = jnp.where(m, neg, _NEG)
            mx = jnp.max(t, axis=1, keepdims=True)
            mval = jnp.where(m, jnp.broadcast_to(mx, neg.shape), mval)
        e = jnp.exp(neg - mval)
        gk = jax.lax.broadcasted_iota(jnp.int32, (_VPAD, _VPAD), 0) // 32
        gl = jax.lax.broadcasted_iota(jnp.int32, (_VPAD, _VPAD), 1) // 32
        seg_ones = (gk == gl).astype(jnp.float32)
        esum = jnp.dot(e, seg_ones, preferred_element_type=jnp.float32)
        q = e / esum
        qb = q.astype(jnp.bfloat16)
        q_ref[i] = qb
        qtx_ref[...] += jax.lax.dot_general(
            qb, x.astype(jnp.bfloat16), _DIMS_CC,
            preferred_element_type=jnp.float32)
        qs_ref[...] += jax.lax.dot_general(
            q, jnp.ones((_BN, 8), jnp.float32), _DIMS_CC,
            preferred_element_type=jnp.float32)
        o_ref[...] = x

    @pl.when((c == 1) & (i == 0))
    def _vertex():
        qsum = qs_ref[...][:, 0:1] + 1e-6
        z = (qtx_ref[...] / qsum - wpk_ref[...]) / spk_ref[...]
        z2_ref[...] = jnp.zeros_like(z2_ref)
        for s, (lo, hi) in enumerate(_SEGS):
            zs = z[lo:hi, :]
            zs = zs / (jnp.sqrt(jnp.sum(zs * zs, axis=1, keepdims=True)) + 1e-6)
            g = jax.lax.dot_general(zs, zs, _DIMS_RR,
                                    preferred_element_type=jnp.float32)
            g = g - jnp.max(g, axis=1, keepdims=True)
            a = jnp.exp(g)
            a = a / jnp.sum(a, axis=1, keepdims=True)
            az = jnp.dot(a, zs, preferred_element_type=jnp.float32)
            wg = (g0_ref, g1_ref, g2_ref, g3_ref)[s][...].astype(jnp.bfloat16)
            z2 = jnp.maximum(
                jnp.dot(az.astype(jnp.bfloat16), wg,
                        preferred_element_type=jnp.float32), 0.0)
            z2_ref[lo:hi, _D * s:_D * (s + 1)] = z2.astype(jnp.bfloat16)

    @pl.when(c >= 1)
    def _project():
        o_ref[...] = jnp.dot(
            q_ref[i], z2_ref[:, pl.ds((c - 1) * _D, _D)],
            preferred_element_type=jnp.float32)


def kernel(x, anchors0, sigma0, Wg0, anchors1, sigma1, Wg1,
           anchors2, sigma2, Wg2, anchors3, sigma3, Wg3):
    n, d = x.shape
    nb = n // _BN

    def _full(shape):
        nd = len(shape)
        return pl.BlockSpec(shape, lambda c, i, _nd=nd: (0,) * _nd)

    out = pl.pallas_call(
        _gcu_body,
        grid=(5, nb),
        in_specs=[
            pl.BlockSpec((_BN, d),
                         lambda c, i: (jnp.where(c == 0, i, nb - 1), 0)),
            _full(anchors0.shape), _full(sigma0.shape), _full(Wg0.shape),
            _full(anchors1.shape), _full(sigma1.shape), _full(Wg1.shape),
            _full(anchors2.shape), _full(sigma2.shape), _full(Wg2.shape),
            _full(anchors3.shape), _full(sigma3.shape), _full(Wg3.shape),
        ],
        out_specs=pl.BlockSpec((_BN, d), lambda c, i: (i, c)),
        out_shape=jax.ShapeDtypeStruct((n, 5 * d), jnp.float32),
        scratch_shapes=[
            pltpu.VMEM((n // _BN, _BN, _VPAD), jnp.bfloat16),
            pltpu.VMEM((_VPAD, d), jnp.float32),
            pltpu.VMEM((_VPAD, 8), jnp.float32),
            pltpu.VMEM((_VPAD, 4 * d), jnp.bfloat16),
            pltpu.VMEM((_VPAD, d), jnp.float32),
            pltpu.VMEM((_VPAD, d), jnp.float32),
            pltpu.VMEM((_VPAD, d), jnp.float32),
            pltpu.VMEM((_VPAD, d), jnp.float32),
            pltpu.VMEM((8, _VPAD), jnp.float32),
        ],
    )(x, anchors0, sigma0, Wg0, anchors1, sigma1, Wg1,
      anchors2, sigma2, Wg2, anchors3, sigma3, Wg3)
    return out


# speedup vs baseline: 1.1940x; 1.1940x over previous
import jax
import jax.numpy as jnp
from jax.experimental import pallas as pl
from jax.experimental.pallas import tpu as pltpu

_VS = (2, 4, 8, 32)
_VPAD = 128
_D = 512
_BN = 2000
_SEGS = tuple((32 * s, 32 * s + v) for s, v in enumerate(_VS))
_NEG = -1e30
_DIMS_RR = (((1,), (1,)), ((), ()))
_DIMS_CC = (((0,), (0,)), ((), ()))
_HALF = 1280


def _gcu_body(x_ref, a0_ref, s0_ref, g0_ref, a1_ref, s1_ref, g1_ref,
              a2_ref, s2_ref, g2_ref, a3_ref, s3_ref, g3_ref,
              o_ref, q_ref, xs_ref, qtx_ref, qs_ref, z2_ref,
              wpk_ref, spk_ref, nh_ref, wi_ref, t3_ref):
    c = pl.program_id(0)
    i = pl.program_id(1)

    @pl.when((c == 0) & (i == 0))
    def _prep():
        spk_ref[...] = jnp.ones_like(spk_ref)
        wpk_ref[...] = jnp.zeros_like(wpk_ref)
        for (lo, hi), a_ref, s_ref in ((_SEGS[0], a0_ref, s0_ref),
                                       (_SEGS[1], a1_ref, s1_ref),
                                       (_SEGS[2], a2_ref, s2_ref),
                                       (_SEGS[3], a3_ref, s3_ref)):
            wpk_ref[lo:hi, :] = a_ref[...]
            spk_ref[lo:hi, :] = jnp.abs(s_ref[...]) + 1e-4
        sig = spk_ref[...]
        w = wpk_ref[...]
        inv2 = 1.0 / (sig * sig)
        nh_ref[...] = -0.5 * inv2
        wi_ref[...] = w * inv2
        t3 = jax.lax.dot_general(
            jnp.ones((8, _D), jnp.float32), -0.5 * (w * w) * inv2, _DIMS_RR,
            preferred_element_type=jnp.float32)
        lane = jax.lax.broadcasted_iota(jnp.int32, (8, _VPAD), 1)
        within = lane % 32
        group = lane // 32
        vlim = jnp.where(group == 0, _VS[0],
                         jnp.where(group == 1, _VS[1],
                                   jnp.where(group == 2, _VS[2], _VS[3])))
        t3_ref[...] = jnp.where(within >= vlim, _NEG, t3)
        qtx_ref[...] = jnp.zeros_like(qtx_ref)
        qs_ref[...] = jnp.zeros_like(qs_ref)

    @pl.when(c == 0)
    def _assign():
        x = x_ref[...]
        t1 = jax.lax.dot_general(x * x, nh_ref[...], _DIMS_RR,
                                 preferred_element_type=jnp.float32)
        t2 = jax.lax.dot_general(x, wi_ref[...], _DIMS_RR,
                                 preferred_element_type=jnp.float32)
        neg = t1 + t2 + t3_ref[0:1, :]
        lane = jax.lax.broadcasted_iota(jnp.int32, neg.shape, 1)
        mval = jnp.full_like(neg, 1e30)
        for lo, hi in _SEGS:
            m = (lane >= lo) & (lane < hi)
            t = jnp.where(m, neg, _NEG)
            mx = jnp.max(t, axis=1, keepdims=True)
            mval = jnp.where(m, jnp.broadcast_to(mx, neg.shape), mval)
        e = jnp.exp(neg - mval)
        gk = jax.lax.broadcasted_iota(jnp.int32, (_VPAD, _VPAD), 0) // 32
        gl = jax.lax.broadcasted_iota(jnp.int32, (_VPAD, _VPAD), 1) // 32
        seg_ones = (gk == gl).astype(jnp.float32)
        esum = jnp.dot(e, seg_ones, preferred_element_type=jnp.float32)
        q = e / esum
        qb = q.astype(jnp.bfloat16)
        q_ref[i] = qb
        qtx_ref[...] += jax.lax.dot_general(
            qb, x.astype(jnp.bfloat16), _DIMS_CC,
            preferred_element_type=jnp.float32)
        qs_ref[...] += jax.lax.dot_general(
            q, jnp.ones((_BN, 8), jnp.float32), _DIMS_CC,
            preferred_element_type=jnp.float32)
        xs_ref[i] = x.astype(jnp.bfloat16)

    @pl.when((c == 1) & (i == 0))
    def _vertex():
        qsum = qs_ref[...][:, 0:1] + 1e-6
        z = (qtx_ref[...] / qsum - wpk_ref[...]) / spk_ref[...]
        z2_ref[...] = jnp.zeros_like(z2_ref)
        for s, (lo, hi) in enumerate(_SEGS):
            zs = z[lo:hi, :]
            zs = zs / (jnp.sqrt(jnp.sum(zs * zs, axis=1, keepdims=True)) + 1e-6)
            g = jax.lax.dot_general(zs, zs, _DIMS_RR,
                                    preferred_element_type=jnp.float32)
            g = g - jnp.max(g, axis=1, keepdims=True)
            a = jnp.exp(g)
            a = a / jnp.sum(a, axis=1, keepdims=True)
            az = jnp.dot(a, zs, preferred_element_type=jnp.float32)
            wg = (g0_ref, g1_ref, g2_ref, g3_ref)[s][...].astype(jnp.bfloat16)
            z2 = jnp.maximum(
                jnp.dot(az.astype(jnp.bfloat16), wg,
                        preferred_element_type=jnp.float32), 0.0)
            z2_ref[lo:hi, _D * s:_D * (s + 1)] = z2.astype(jnp.bfloat16)

    @pl.when(c == 1)
    def _project_lo():
        o_ref[:, 0:_D] = xs_ref[i].astype(jnp.float32)
        o_ref[:, _D:] = jnp.dot(
            q_ref[i], z2_ref[:, 0:_HALF - _D],
            preferred_element_type=jnp.float32)

    @pl.when(c == 2)
    def _project_hi():
        o_ref[...] = jnp.dot(
            q_ref[i], z2_ref[:, _HALF - _D:4 * _D],
            preferred_element_type=jnp.float32)


def kernel(x, anchors0, sigma0, Wg0, anchors1, sigma1, Wg1,
           anchors2, sigma2, Wg2, anchors3, sigma3, Wg3):
    n, d = x.shape
    nb = n // _BN

    def _full(shape):
        nd = len(shape)
        return pl.BlockSpec(shape, lambda c, i, _nd=nd: (0,) * _nd)

    out = pl.pallas_call(
        _gcu_body,
        grid=(3, nb),
        in_specs=[
            pl.BlockSpec((_BN, d),
                         lambda c, i: (jnp.where(c == 0, i, nb - 1), 0)),
            _full(anchors0.shape), _full(sigma0.shape), _full(Wg0.shape),
            _full(anchors1.shape), _full(sigma1.shape), _full(Wg1.shape),
            _full(anchors2.shape), _full(sigma2.shape), _full(Wg2.shape),
            _full(anchors3.shape), _full(sigma3.shape), _full(Wg3.shape),
        ],
        out_specs=pl.BlockSpec(
            (_BN, _HALF),
            lambda c, i: (jnp.where(c == 0, 0, i), jnp.maximum(c - 1, 0))),
        out_shape=jax.ShapeDtypeStruct((n, 5 * d), jnp.float32),
        scratch_shapes=[
            pltpu.VMEM((n // _BN, _BN, _VPAD), jnp.bfloat16),
            pltpu.VMEM((n // _BN, _BN, _D), jnp.bfloat16),
            pltpu.VMEM((_VPAD, d), jnp.float32),
            pltpu.VMEM((_VPAD, 8), jnp.float32),
            pltpu.VMEM((_VPAD, 4 * d), jnp.bfloat16),
            pltpu.VMEM((_VPAD, d), jnp.float32),
            pltpu.VMEM((_VPAD, d), jnp.float32),
            pltpu.VMEM((_VPAD, d), jnp.float32),
            pltpu.VMEM((_VPAD, d), jnp.float32),
            pltpu.VMEM((8, _VPAD), jnp.float32),
        ],
    )(x, anchors0, sigma0, Wg0, anchors1, sigma1, Wg1,
      anchors2, sigma2, Wg2, anchors3, sigma3, Wg3)
    return out


# speedup vs baseline: 1.2330x; 1.0327x over previous
import jax
import jax.numpy as jnp
from jax.experimental import pallas as pl
from jax.experimental.pallas import tpu as pltpu

_VS = (2, 4, 8, 32)
_VPAD = 128
_D = 512
_BN = 2000
_SEGS = tuple((32 * s, 32 * s + v) for s, v in enumerate(_VS))
_NEG = -1e30
_DIMS_RR = (((1,), (1,)), ((), ()))
_DIMS_CC = (((0,), (0,)), ((), ()))


def _gcu_body(x_ref, a0_ref, s0_ref, g0_ref, a1_ref, s1_ref, g1_ref,
              a2_ref, s2_ref, g2_ref, a3_ref, s3_ref, g3_ref,
              o_ref, q_ref, qtx_ref, qs_ref, z2_ref,
              wpk_ref, spk_ref, nh_ref, wi_ref, t3_ref):
    c = pl.program_id(0)
    i = pl.program_id(1)

    @pl.when((c == 0) & (i == 0))
    def _prep():
        spk_ref[...] = jnp.ones_like(spk_ref)
        wpk_ref[...] = jnp.zeros_like(wpk_ref)
        for (lo, hi), a_ref, s_ref in ((_SEGS[0], a0_ref, s0_ref),
                                       (_SEGS[1], a1_ref, s1_ref),
                                       (_SEGS[2], a2_ref, s2_ref),
                                       (_SEGS[3], a3_ref, s3_ref)):
            wpk_ref[lo:hi, :] = a_ref[...]
            spk_ref[lo:hi, :] = jnp.abs(s_ref[...]) + 1e-4
        sig = spk_ref[...]
        w = wpk_ref[...]
        inv2 = 1.0 / (sig * sig)
        nh_ref[...] = -0.5 * inv2
        wi_ref[...] = w * inv2
        t3 = jax.lax.dot_general(
            jnp.ones((8, _D), jnp.float32), -0.5 * (w * w) * inv2, _DIMS_RR,
            preferred_element_type=jnp.float32)
        lane = jax.lax.broadcasted_iota(jnp.int32, (8, _VPAD), 1)
        within = lane % 32
        group = lane // 32
        vlim = jnp.where(group == 0, _VS[0],
                         jnp.where(group == 1, _VS[1],
                                   jnp.where(group == 2, _VS[2], _VS[3])))
        t3_ref[...] = jnp.where(within >= vlim, _NEG, t3)
        qtx_ref[...] = jnp.zeros_like(qtx_ref)
        qs_ref[...] = jnp.zeros_like(qs_ref)

    @pl.when(c == 0)
    def _assign():
        x = x_ref[...]
        t1 = jax.lax.dot_general(x * x, nh_ref[...], _DIMS_RR,
                                 preferred_element_type=jnp.float32)
        t2 = jax.lax.dot_general(x, wi_ref[...], _DIMS_RR,
                                 preferred_element_type=jnp.float32)
        neg = t1 + t2 + t3_ref[0:1, :]
        lane = jax.lax.broadcasted_iota(jnp.int32, neg.shape, 1)
        mval = jnp.full_like(neg, 1e30)
        for lo, hi in _SEGS:
            m = (lane >= lo) & (lane < hi)
            t = jnp.where(m, neg, _NEG)
            mx = jnp.max(t, axis=1, keepdims=True)
            mval = jnp.where(m, jnp.broadcast_to(mx, neg.shape), mval)
        e = jnp.exp(neg - mval)
        gk = jax.lax.broadcasted_iota(jnp.int32, (_VPAD, _VPAD), 0) // 32
        gl = jax.lax.broadcasted_iota(jnp.int32, (_VPAD, _VPAD), 1) // 32
        seg_ones = (gk == gl).astype(jnp.float32)
        esum = jnp.dot(e, seg_ones, preferred_element_type=jnp.float32)
        q = e / esum
        qb = q.astype(jnp.bfloat16)
        q_ref[i] = qb
        qtx_ref[...] += jax.lax.dot_general(
            qb, x.astype(jnp.bfloat16), _DIMS_CC,
            preferred_element_type=jnp.float32)
        qs_ref[...] += jax.lax.dot_general(
            q, jnp.ones((_BN, 8), jnp.float32), _DIMS_CC,
            preferred_element_type=jnp.float32)
        o_ref[...] = x

    @pl.when((c == 1) & (i == 0))
    def _vertex():
        qsum = qs_ref[...][:, 0:1] + 1e-6
        z = (qtx_ref[...] / qsum - wpk_ref[...]) / spk_ref[...]
        z2_ref[...] = jnp.zeros_like(z2_ref)
        for s, (lo, hi) in enumerate(_SEGS):
            zs = z[lo:hi, :]
            zs = zs / (jnp.sqrt(jnp.sum(zs * zs, axis=1, keepdims=True)) + 1e-6)
            g = jax.lax.dot_general(zs, zs, _DIMS_RR,
                                    preferred_element_type=jnp.float32)
            g = g - jnp.max(g, axis=1, keepdims=True)
            a = jnp.exp(g)
            a = a / jnp.sum(a, axis=1, keepdims=True)
            az = jnp.dot(a, zs, preferred_element_type=jnp.float32)
            wg = (g0_ref, g1_ref, g2_ref, g3_ref)[s][...].astype(jnp.bfloat16)
            z2 = jnp.maximum(
                jnp.dot(az.astype(jnp.bfloat16), wg,
                        preferred_element_type=jnp.float32), 0.0)
            z2_ref[lo:hi, _D * s:_D * (s + 1)] = z2.astype(jnp.bfloat16)

    @pl.when(c >= 1)
    def _project():
        o_ref[...] = jnp.dot(
            q_ref[i], z2_ref[:, pl.ds((c - 1) * _D, _D)],
            preferred_element_type=jnp.float32)


def kernel(x, anchors0, sigma0, Wg0, anchors1, sigma1, Wg1,
           anchors2, sigma2, Wg2, anchors3, sigma3, Wg3):
    n, d = x.shape
    nb = n // _BN

    def _full(shape):
        nd = len(shape)
        return pl.BlockSpec(shape, lambda c, i, _nd=nd: (0,) * _nd)

    out = pl.pallas_call(
        _gcu_body,
        grid=(5, nb),
        in_specs=[
            pl.BlockSpec((_BN, d),
                         lambda c, i: (jnp.where(c == 0, i, nb - 1), 0)),
            _full(anchors0.shape), _full(sigma0.shape), _full(Wg0.shape),
            _full(anchors1.shape), _full(sigma1.shape), _full(Wg1.shape),
            _full(anchors2.shape), _full(sigma2.shape), _full(Wg2.shape),
            _full(anchors3.shape), _full(sigma3.shape), _full(Wg3.shape),
        ],
        out_specs=pl.BlockSpec((_BN, d), lambda c, i: (i, c)),
        out_shape=jax.ShapeDtypeStruct((n, 5 * d), jnp.float32),
        scratch_shapes=[
            pltpu.VMEM((n // _BN, _BN, _VPAD), jnp.bfloat16),
            pltpu.VMEM((_VPAD, d), jnp.float32),
            pltpu.VMEM((_VPAD, 8), jnp.float32),
            pltpu.VMEM((_VPAD, 4 * d), jnp.bfloat16),
            pltpu.VMEM((_VPAD, d), jnp.float32),
            pltpu.VMEM((_VPAD, d), jnp.float32),
            pltpu.VMEM((_VPAD, d), jnp.float32),
            pltpu.VMEM((_VPAD, d), jnp.float32),
            pltpu.VMEM((8, _VPAD), jnp.float32),
        ],
    )(x, anchors0, sigma0, Wg0, anchors1, sigma1, Wg1,
      anchors2, sigma2, Wg2, anchors3, sigma3, Wg3)
    return out
